# baseline (speedup 1.0000x reference)
import functools

import jax
import jax.numpy as jnp
from jax import lax
from jax.experimental import pallas as pl
from jax.experimental.pallas import tpu as pltpu
from jax.experimental.pallas import tpu_sc as plsc

VOCAB = 1000000
EMB_DIM = 64

NC = 2
NS = 16
NW = NC * NS

TW = 128
NBUF = 2


def _emb_body(S, TB, idx_hbm, table_hbm, out_hbm, idx_v, rows_v, tiles_v,
              g0, g1, w0, w1):
  gsem = (g0, g1)
  wsem = (w0, w1)
  wid = lax.axis_index("s") * NC + lax.axis_index("c")
  t0 = wid * (TB * TW)
  n_units = S * TB

  pltpu.sync_copy(idx_hbm.at[:, pl.ds(t0, TB * TW)], idx_v)

  def fire_gather(u, b):
    s = u // TB
    q = u % TB
    pltpu.async_copy(
        table_hbm.at[idx_v.at[s, pl.ds(q * TW, TW)]],
        rows_v.at[b],
        gsem[b],
    )

  def wait_gather(b):
    pltpu.make_async_copy(
        table_hbm.at[pl.ds(0, TW)], rows_v.at[b], gsem[b]).wait()

  def fire_write(u, b):
    s = u // TB
    t_hi = wid * TB + u % TB
    for ehi in range(8):
      pltpu.async_copy(
          tiles_v.at[b, ehi], out_hbm.at[s, ehi, t_hi], wsem[b])

  def wait_write(b):
    for ehi in range(8):
      pltpu.make_async_copy(
          tiles_v.at[b, ehi], out_hbm.at[0, ehi, 0], wsem[b]).wait()

  iota = lax.iota(jnp.int32, 16)

  def transpose(b):
    rows = rows_v.at[b]

    def e_body(e, _):
      ehi = e // 8
      elo = e % 8
      ev = iota * 0 + e
      for tb in range(8):
        vec = plsc.load_gather(rows, [iota + tb * 16, ev])
        tiles_v[b, ehi, elo, pl.ds(tb * 16, 16)] = vec
      return 0

    lax.fori_loop(0, EMB_DIM, e_body, 0)

  fire_gather(0, 0)

  def pair(g, _):
    for b in range(NBUF):
      u = g * NBUF + b

      @pl.when(u + 1 < n_units)
      def _():
        fire_gather(u + 1, 1 - b)

      wait_gather(b)

      @pl.when(u >= NBUF)
      def _():
        wait_write(b)

      transpose(b)
      fire_write(u, b)

    return 0

  lax.fori_loop(0, n_units // NBUF, pair, 0)
  for b in range(NBUF):
    wait_write(b)


def kernel(indices, table):
  T, S = indices.shape
  assert T % (NW * TW) == 0 and EMB_DIM == 64
  TB = T // (NW * TW)
  idx_t = indices.T.astype(jnp.int32)

  mesh = plsc.VectorSubcoreMesh(core_axis_name="c", subcore_axis_name="s")
  grab = pl.kernel(
      functools.partial(_emb_body, S, TB),
      out_type=jax.ShapeDtypeStruct((S, 8, T // TW, 8, TW), jnp.float32),
      mesh=mesh,
      scratch_types=[
          pltpu.VMEM((S, TB * TW), jnp.int32),
          pltpu.VMEM((NBUF, TW, EMB_DIM), jnp.float32),
          pltpu.VMEM((NBUF, 8, 8, TW), jnp.float32),
          pltpu.SemaphoreType.DMA,
          pltpu.SemaphoreType.DMA,
          pltpu.SemaphoreType.DMA,
          pltpu.SemaphoreType.DMA,
      ],
      compiler_params=pltpu.CompilerParams(
          use_tc_tiling_on_sc=False, needs_layout_passes=False),
  )
  q = grab(idx_t, table)
  return q.transpose(2, 4, 0, 1, 3).reshape(T, S, EMB_DIM)


# speedup vs baseline: 1.0005x; 1.0005x over previous
import functools

import jax
import jax.numpy as jnp
from jax import lax
from jax.experimental import pallas as pl
from jax.experimental.pallas import tpu as pltpu
from jax.experimental.pallas import tpu_sc as plsc

VOCAB = 1000000
EMB_DIM = 64

NC = 2
NS = 16
NW = NC * NS

TW = 128
NBUF = 2


def _emb_body(S, TB, idx_hbm, table_hbm, out_hbm, idx_v, rows_v, tiles_v,
              g0, g1, w0, w1):
  gsem = (g0, g1)
  wsem = (w0, w1)
  wid = lax.axis_index("s") * NC + lax.axis_index("c")
  t0 = wid * (TB * TW)
  n_units = S * TB

  pltpu.sync_copy(idx_hbm.at[:, pl.ds(t0, TB * TW)], idx_v)

  def fire_gather(u, b):
    s = u // TB
    q = u % TB
    pltpu.async_copy(
        table_hbm.at[idx_v.at[s, pl.ds(q * TW, TW)]],
        rows_v.at[b],
        gsem[b],
    )

  def wait_gather(b):
    pltpu.make_async_copy(
        table_hbm.at[pl.ds(0, TW)], rows_v.at[b], gsem[b]).wait()

  def fire_write(u, b):
    s = u // TB
    t_hi = wid * TB + u % TB
    for ehi in range(8):
      pltpu.async_copy(
          tiles_v.at[b, pl.ds(ehi * 8, 8)], out_hbm.at[s, ehi, t_hi], wsem[b])

  def wait_write(b):
    for ehi in range(8):
      pltpu.make_async_copy(
          tiles_v.at[b, pl.ds(ehi * 8, 8)], out_hbm.at[0, ehi, 0],
          wsem[b]).wait()

  iota = lax.iota(jnp.int32, 16)
  tvs = [iota + tb * 16 for tb in range(8)]

  def transpose(b):
    rows = rows_v.at[b]

    def e_body(e, _):
      ev = lax.broadcast(e, (16,))
      for tb in range(8):
        tiles_v[b, e, pl.ds(tb * 16, 16)] = plsc.load_gather(
            rows, [tvs[tb], ev])
      return 0

    lax.fori_loop(0, EMB_DIM, e_body, 0, unroll=4)

  fire_gather(0, 0)

  def pair(g, _):
    for b in range(NBUF):
      u = g * NBUF + b

      @pl.when(u + 1 < n_units)
      def _():
        fire_gather(u + 1, 1 - b)

      wait_gather(b)

      @pl.when(u >= NBUF)
      def _():
        wait_write(b)

      transpose(b)
      fire_write(u, b)

    return 0

  lax.fori_loop(0, n_units // NBUF, pair, 0)
  for b in range(NBUF):
    wait_write(b)


def kernel(indices, table):
  T, S = indices.shape
  assert T % (NW * TW) == 0 and EMB_DIM == 64
  TB = T // (NW * TW)
  idx_t = indices.T.astype(jnp.int32)

  mesh = plsc.VectorSubcoreMesh(core_axis_name="c", subcore_axis_name="s")
  grab = pl.kernel(
      functools.partial(_emb_body, S, TB),
      out_type=jax.ShapeDtypeStruct((S, 8, T // TW, 8, TW), jnp.float32),
      mesh=mesh,
      scratch_types=[
          pltpu.VMEM((S, TB * TW), jnp.int32),
          pltpu.VMEM((NBUF, TW, EMB_DIM), jnp.float32),
          pltpu.VMEM((NBUF, EMB_DIM, TW), jnp.float32),
          pltpu.SemaphoreType.DMA,
          pltpu.SemaphoreType.DMA,
          pltpu.SemaphoreType.DMA,
          pltpu.SemaphoreType.DMA,
      ],
      compiler_params=pltpu.CompilerParams(
          use_tc_tiling_on_sc=False, needs_layout_passes=False),
  )
  q = grab(idx_t, table)
  return q.transpose(2, 4, 0, 1, 3).reshape(T, S, EMB_DIM)


# speedup vs baseline: 1.4576x; 1.4569x over previous
import functools

import jax
import jax.numpy as jnp
from jax import lax
from jax.experimental import pallas as pl
from jax.experimental.pallas import tpu as pltpu
from jax.experimental.pallas import tpu_sc as plsc

VOCAB = 1000000
EMB_DIM = 64

NC = 2
NS = 16
NW = NC * NS

TW = 128
NBUF = 2


def _emb_body(S, TB, idx_hbm, table_hbm, out_hbm, idx_v, rows_v, tiles_v,
              g0, g1, w0, w1):
  gsem = (g0, g1)
  wsem = (w0, w1)
  wid = lax.axis_index("s") * NC + lax.axis_index("c")
  t0 = wid * (TB * TW)
  n_units = S * TB

  pltpu.sync_copy(idx_hbm.at[:, pl.ds(t0, TB * TW)], idx_v)

  def fire_gather(u, b):
    s = u // TB
    q = u % TB
    pltpu.async_copy(
        table_hbm.at[idx_v.at[s, pl.ds(q * TW, TW)]],
        rows_v.at[b],
        gsem[b],
    )

  def wait_gather(b):
    pltpu.make_async_copy(
        table_hbm.at[pl.ds(0, TW)], rows_v.at[b], gsem[b]).wait()

  def fire_write(u, b):
    s = u // TB
    t_hi = wid * TB + u % TB
    for ehi in range(8):
      pltpu.async_copy(
          tiles_v.at[b, pl.ds(ehi * 8, 8)], out_hbm.at[s, ehi, t_hi], wsem[b])

  def wait_write(b):
    for ehi in range(8):
      pltpu.make_async_copy(
          tiles_v.at[b, pl.ds(ehi * 8, 8)], out_hbm.at[0, ehi, 0],
          wsem[b]).wait()

  iota = lax.iota(jnp.int32, 16)
  tvs = [iota + tb * 16 for tb in range(8)]

  def transpose(b):
    rows = rows_v.at[b]

    @plsc.parallel_loop(0, EMB_DIM, unroll=4)
    def e_body(e):
      ev = lax.broadcast(e, (16,))
      for tb in range(8):
        tiles_v[b, e, pl.ds(tb * 16, 16)] = plsc.load_gather(
            rows, [tvs[tb], ev])

  fire_gather(0, 0)

  def pair(g, _):
    for b in range(NBUF):
      u = g * NBUF + b

      @pl.when(u + 1 < n_units)
      def _():
        fire_gather(u + 1, 1 - b)

      wait_gather(b)

      @pl.when(u >= NBUF)
      def _():
        wait_write(b)

      transpose(b)
      fire_write(u, b)

    return 0

  lax.fori_loop(0, n_units // NBUF, pair, 0)
  for b in range(NBUF):
    wait_write(b)


def kernel(indices, table):
  T, S = indices.shape
  assert T % (NW * TW) == 0 and EMB_DIM == 64
  TB = T // (NW * TW)
  idx_t = indices.T.astype(jnp.int32)

  mesh = plsc.VectorSubcoreMesh(core_axis_name="c", subcore_axis_name="s")
  grab = pl.kernel(
      functools.partial(_emb_body, S, TB),
      out_type=jax.ShapeDtypeStruct((S, 8, T // TW, 8, TW), jnp.float32),
      mesh=mesh,
      scratch_types=[
          pltpu.VMEM((S, TB * TW), jnp.int32),
          pltpu.VMEM((NBUF, TW, EMB_DIM), jnp.float32),
          pltpu.VMEM((NBUF, EMB_DIM, TW), jnp.float32),
          pltpu.SemaphoreType.DMA,
          pltpu.SemaphoreType.DMA,
          pltpu.SemaphoreType.DMA,
          pltpu.SemaphoreType.DMA,
      ],
      compiler_params=pltpu.CompilerParams(
          use_tc_tiling_on_sc=False, needs_layout_passes=False),
  )
  q = grab(idx_t, table)
  return q.transpose(2, 4, 0, 1, 3).reshape(T, S, EMB_DIM)


# speedup vs baseline: 1.6287x; 1.1174x over previous
import functools

import jax
import jax.numpy as jnp
from jax import lax
from jax.experimental import pallas as pl
from jax.experimental.pallas import tpu as pltpu
from jax.experimental.pallas import tpu_sc as plsc

VOCAB = 1000000
EMB_DIM = 64

NC = 2
NS = 16
NW = NC * NS

TW = 128
NBUF = 2
YS = TW + 5

NCOL = (VOCAB + TW - 1) // TW
FULL_COL = VOCAB // TW


def _fmt_body(tbl_hbm, tail_hbm, lin_hbm, x_v, y_v, g0, g1, w0, w1):
  gsem = (g0, g1)
  wsem = (w0, w1)
  wid = lax.axis_index("s") * NC + lax.axis_index("c")
  n_units = (NCOL + NW - 1) // NW

  iota = lax.iota(jnp.int32, 16)
  vbs = [iota + vb * 16 for vb in range(TW // 16)]

  def fire_in(u, b):
    c = u * NW + wid

    @pl.when(c < FULL_COL)
    def _():
      pltpu.async_copy(tbl_hbm.at[:, pl.ds(c * TW, TW)], x_v.at[b], gsem[b])

    @pl.when(c == FULL_COL)
    def _():
      pltpu.async_copy(tail_hbm, x_v.at[b], gsem[b])

  def wait_in(u, b):
    c = u * NW + wid

    @pl.when(c < FULL_COL)
    def _():
      pltpu.make_async_copy(
          tbl_hbm.at[:, pl.ds(0, TW)], x_v.at[b], gsem[b]).wait()

    @pl.when(c == FULL_COL)
    def _():
      pltpu.make_async_copy(tail_hbm, x_v.at[b], gsem[b]).wait()

  def transpose(b):
    x = x_v.at[b]
    y = y_v.at[b]

    @plsc.parallel_loop(0, EMB_DIM, unroll=4)
    def e_body(e):
      ev = lax.broadcast(e, (16,))
      for vb in range(TW // 16):
        plsc.store_scatter(y, [vbs[vb], ev], x[e, pl.ds(vb * 16, 16)])

  def fire_out(u, b):
    c = u * NW + wid

    @pl.when(c < FULL_COL)
    def _():
      pltpu.async_copy(
          y_v.at[b, :, pl.ds(0, TW)],
          lin_hbm.at[pl.ds(c * TW, TW)], wsem[b])

    @pl.when(c == FULL_COL)
    def _():
      pltpu.async_copy(
          y_v.at[b, pl.ds(0, TW // 2), pl.ds(0, TW)],
          lin_hbm.at[pl.ds(FULL_COL * TW, TW // 2)], wsem[b])

  def wait_out(u, b):
    c = u * NW + wid

    @pl.when(c < FULL_COL)
    def _():
      pltpu.make_async_copy(
          y_v.at[b, :, pl.ds(0, TW)],
          lin_hbm.at[pl.ds(0, TW)], wsem[b]).wait()

    @pl.when(c == FULL_COL)
    def _():
      pltpu.make_async_copy(
          y_v.at[b, pl.ds(0, TW // 2), pl.ds(0, TW)],
          lin_hbm.at[pl.ds(0, TW // 2)], wsem[b]).wait()

  fire_in(0, 0)

  def pair(g, _):
    for b in range(NBUF):
      u = g * NBUF + b

      @pl.when((u + 1) * NW + wid < NCOL)
      def _():
        fire_in(u + 1, 1 - b)

      @pl.when(u * NW + wid < NCOL)
      def _():
        wait_in(u, b)

        @pl.when(u >= NBUF)
        def _():
          wait_out(u - NBUF, b)

        transpose(b)
        fire_out(u, b)

    return 0

  n_outer = (n_units + NBUF - 1) // NBUF
  lax.fori_loop(0, n_outer, pair, 0)
  lv = (NCOL - 1 - wid) // NW
  for k in range(NBUF):
    u = lv - k
    for bb in range(NBUF):
      @pl.when(lax.rem(u, NBUF) == bb)
      def _():
        wait_out(u, bb)


def _emb_body(S, TB, idx_hbm, table_hbm, out_hbm, idx_v, rows_v, tiles_v,
              g0, g1, w0, w1):
  gsem = (g0, g1)
  wsem = (w0, w1)
  wid = lax.axis_index("s") * NC + lax.axis_index("c")
  t0 = wid * (TB * TW)
  n_units = S * TB

  pltpu.sync_copy(idx_hbm.at[:, pl.ds(t0, TB * TW)], idx_v)

  def fire_gather(u, b):
    s = u // TB
    q = u % TB
    pltpu.async_copy(
        table_hbm.at[idx_v.at[s, pl.ds(q * TW, TW)]],
        rows_v.at[b],
        gsem[b],
    )

  def wait_gather(b):
    pltpu.make_async_copy(
        table_hbm.at[pl.ds(0, TW)], rows_v.at[b], gsem[b]).wait()

  def fire_write(u, b):
    s = u // TB
    t_hi = wid * TB + u % TB
    for ehi in range(8):
      pltpu.async_copy(
          tiles_v.at[b, pl.ds(ehi * 8, 8), pl.ds(0, TW)],
          out_hbm.at[s, ehi, t_hi], wsem[b])

  def wait_write(b):
    for ehi in range(8):
      pltpu.make_async_copy(
          tiles_v.at[b, pl.ds(ehi * 8, 8), pl.ds(0, TW)],
          out_hbm.at[0, ehi, 0], wsem[b]).wait()

  iota = lax.iota(jnp.int32, 16)
  evs = [iota + eb * 16 for eb in range(EMB_DIM // 16)]

  def transpose(b):
    rows = rows_v.at[b]
    tiles = tiles_v.at[b]

    @plsc.parallel_loop(0, TW, unroll=4)
    def t_body(t):
      cv = lax.broadcast(t, (16,))
      for eb in range(EMB_DIM // 16):
        plsc.store_scatter(
            tiles, [evs[eb], cv], rows[t, pl.ds(eb * 16, 16)])

  fire_gather(0, 0)

  def pair(g, _):
    for b in range(NBUF):
      u = g * NBUF + b

      @pl.when(u + 1 < n_units)
      def _():
        fire_gather(u + 1, 1 - b)

      wait_gather(b)

      @pl.when(u >= NBUF)
      def _():
        wait_write(b)

      transpose(b)
      fire_write(u, b)

    return 0

  lax.fori_loop(0, n_units // NBUF, pair, 0)
  for b in range(NBUF):
    wait_write(b)


def kernel(indices, table):
  T, S = indices.shape
  assert T % (NW * TW) == 0 and EMB_DIM == 64
  TB = T // (NW * TW)
  idx_t = indices.T.astype(jnp.int32)

  mesh = plsc.VectorSubcoreMesh(core_axis_name="c", subcore_axis_name="s")

  fmt = pl.kernel(
      _fmt_body,
      out_type=jax.ShapeDtypeStruct((VOCAB, TW), jnp.float32),
      mesh=mesh,
      scratch_types=[
          pltpu.VMEM((NBUF, EMB_DIM, TW), jnp.float32),
          pltpu.VMEM((NBUF, TW, YS), jnp.float32),
          pltpu.SemaphoreType.DMA,
          pltpu.SemaphoreType.DMA,
          pltpu.SemaphoreType.DMA,
          pltpu.SemaphoreType.DMA,
      ],
      compiler_params=pltpu.CompilerParams(
          use_tc_tiling_on_sc=True, needs_layout_passes=False),
  )
  tail = jnp.pad(table[FULL_COL * TW:].T, ((0, 0), (0, TW - (VOCAB - FULL_COL * TW))))
  lin = fmt(table.T, tail)

  grab = pl.kernel(
      functools.partial(_emb_body, S, TB),
      out_type=jax.ShapeDtypeStruct((S, 8, T // TW, 8, TW), jnp.float32),
      mesh=mesh,
      scratch_types=[
          pltpu.VMEM((S, TB * TW), jnp.int32),
          pltpu.VMEM((NBUF, TW, TW), jnp.float32),
          pltpu.VMEM((NBUF, EMB_DIM, YS), jnp.float32),
          pltpu.SemaphoreType.DMA,
          pltpu.SemaphoreType.DMA,
          pltpu.SemaphoreType.DMA,
          pltpu.SemaphoreType.DMA,
      ],
      compiler_params=pltpu.CompilerParams(
          use_tc_tiling_on_sc=False, needs_layout_passes=False),
  )
  q = grab(idx_t, lin)
  return q.transpose(2, 4, 0, 1, 3).reshape(T, S, EMB_DIM)


# speedup vs baseline: 3.2869x; 2.0181x over previous
import functools

import jax
import jax.numpy as jnp
from jax import lax
from jax.experimental import pallas as pl
from jax.experimental.pallas import tpu as pltpu
from jax.experimental.pallas import tpu_sc as plsc

VOCAB = 1000000
EMB_DIM = 64

NC = 2
NS = 16
NW = NC * NS

TW = 128
NBUF = 2
YS = TW + 5

NCOL = (VOCAB + TW - 1) // TW
FULL_COL = VOCAB // TW


def _fmt_body(tbl_hbm, tail_hbm, lin_hbm, x_v, y_v, g0, g1, w0, w1):
  gsem = (g0, g1)
  wsem = (w0, w1)
  wid = lax.axis_index("s") * NC + lax.axis_index("c")
  n_units = (NCOL + NW - 1) // NW

  def fire_in(u, b):
    c = u * NW + wid

    @pl.when(c < FULL_COL)
    def _():
      pltpu.async_copy(tbl_hbm.at[:, pl.ds(c * TW, TW)], x_v.at[b], gsem[b])

    @pl.when(c == FULL_COL)
    def _():
      pltpu.async_copy(tail_hbm, x_v.at[b], gsem[b])

  def wait_in(u, b):
    c = u * NW + wid

    @pl.when(c < FULL_COL)
    def _():
      pltpu.make_async_copy(
          tbl_hbm.at[:, pl.ds(0, TW)], x_v.at[b], gsem[b]).wait()

    @pl.when(c == FULL_COL)
    def _():
      pltpu.make_async_copy(tail_hbm, x_v.at[b], gsem[b]).wait()

  iota16 = lax.iota(jnp.int32, 16)
  rots = [(iota16 + k) & 15 for k in range(16)]

  def transpose(b):
    x = x_v.at[b]
    y = y_v.at[b]

    @plsc.parallel_loop(0, (EMB_DIM // 16) * (TW // 16), unroll=2)
    def blk_body(i):
      vb16 = (i & 7) * 16
      eb16 = (i >> 3) * 16
      ev = iota16 + eb16
      for k in range(16):
        vv = rots[k] + vb16
        plsc.store_scatter(y, [vv, ev], plsc.load_gather(x, [ev, vv]))

  def fire_out(u, b):
    c = u * NW + wid

    @pl.when(c < FULL_COL)
    def _():
      pltpu.async_copy(
          y_v.at[b], lin_hbm.at[pl.ds(c * TW, TW)], wsem[b])

    @pl.when(c == FULL_COL)
    def _():
      pltpu.async_copy(
          y_v.at[b, pl.ds(0, TW // 2)],
          lin_hbm.at[pl.ds(FULL_COL * TW, TW // 2)], wsem[b])

  def wait_out(u, b):
    c = u * NW + wid

    @pl.when(c < FULL_COL)
    def _():
      pltpu.make_async_copy(
          y_v.at[b], lin_hbm.at[pl.ds(0, TW)], wsem[b]).wait()

    @pl.when(c == FULL_COL)
    def _():
      pltpu.make_async_copy(
          y_v.at[b, pl.ds(0, TW // 2)],
          lin_hbm.at[pl.ds(0, TW // 2)], wsem[b]).wait()

  fire_in(0, 0)

  def pair(g, _):
    for b in range(NBUF):
      u = g * NBUF + b

      @pl.when((u + 1) * NW + wid < NCOL)
      def _():
        fire_in(u + 1, 1 - b)

      @pl.when(u * NW + wid < NCOL)
      def _():
        wait_in(u, b)

        @pl.when(u >= NBUF)
        def _():
          wait_out(u - NBUF, b)

        transpose(b)
        fire_out(u, b)

    return 0

  n_outer = (n_units + NBUF - 1) // NBUF
  lax.fori_loop(0, n_outer, pair, 0)
  lv = (NCOL - 1 - wid) // NW
  for k in range(NBUF):
    u = lv - k
    for bb in range(NBUF):
      @pl.when(lax.rem(u, NBUF) == bb)
      def _():
        wait_out(u, bb)


def _emb_body(S, TB, idx_hbm, table_hbm, out_hbm, idx_v, rows_v, tiles_v,
              g0, g1, w0, w1):
  gsem = (g0, g1)
  wsem = (w0, w1)
  wid = lax.axis_index("s") * NC + lax.axis_index("c")
  t0 = wid * (TB * TW)
  n_units = S * TB

  pltpu.sync_copy(idx_hbm.at[:, pl.ds(t0, TB * TW)], idx_v)

  def fire_gather(u, b):
    s = u // TB
    q = u % TB
    pltpu.async_copy(
        table_hbm.at[idx_v.at[s, pl.ds(q * TW, TW)]],
        rows_v.at[b],
        gsem[b],
    )

  def wait_gather(b):
    pltpu.make_async_copy(
        table_hbm.at[pl.ds(0, TW)], rows_v.at[b], gsem[b]).wait()

  def fire_write(u, b):
    s = u // TB
    t_hi = wid * TB + u % TB
    for ehi in range(8):
      pltpu.async_copy(
          tiles_v.at[b, pl.ds(ehi * 8, 8), pl.ds(0, TW)],
          out_hbm.at[s, ehi, t_hi], wsem[b])

  def wait_write(b):
    for ehi in range(8):
      pltpu.make_async_copy(
          tiles_v.at[b, pl.ds(ehi * 8, 8), pl.ds(0, TW)],
          out_hbm.at[0, ehi, 0], wsem[b]).wait()

  iota = lax.iota(jnp.int32, 16)
  evs = [iota + eb * 16 for eb in range(EMB_DIM // 16)]

  def transpose(b):
    rows = rows_v.at[b]
    tiles = tiles_v.at[b]

    @plsc.parallel_loop(0, TW, unroll=4)
    def t_body(t):
      cv = lax.broadcast(t, (16,))
      for eb in range(EMB_DIM // 16):
        plsc.store_scatter(
            tiles, [evs[eb], cv], rows[t, pl.ds(eb * 16, 16)])

  fire_gather(0, 0)

  def pair(g, _):
    for b in range(NBUF):
      u = g * NBUF + b

      @pl.when(u + 1 < n_units)
      def _():
        fire_gather(u + 1, 1 - b)

      wait_gather(b)

      @pl.when(u >= NBUF)
      def _():
        wait_write(b)

      transpose(b)
      fire_write(u, b)

    return 0

  lax.fori_loop(0, n_units // NBUF, pair, 0)
  for b in range(NBUF):
    wait_write(b)


def kernel(indices, table):
  T, S = indices.shape
  assert T % (NW * TW) == 0 and EMB_DIM == 64
  TB = T // (NW * TW)
  idx_t = indices.T.astype(jnp.int32)

  mesh = plsc.VectorSubcoreMesh(core_axis_name="c", subcore_axis_name="s")

  fmt = pl.kernel(
      _fmt_body,
      out_type=jax.ShapeDtypeStruct((VOCAB, TW), jnp.float32),
      mesh=mesh,
      scratch_types=[
          pltpu.VMEM((NBUF, EMB_DIM, TW), jnp.float32),
          pltpu.VMEM((NBUF, TW, TW), jnp.float32),
          pltpu.SemaphoreType.DMA,
          pltpu.SemaphoreType.DMA,
          pltpu.SemaphoreType.DMA,
          pltpu.SemaphoreType.DMA,
      ],
      compiler_params=pltpu.CompilerParams(
          use_tc_tiling_on_sc=True, needs_layout_passes=False),
  )
  tail = jnp.pad(table[FULL_COL * TW:].T, ((0, 0), (0, TW - (VOCAB - FULL_COL * TW))))
  lin = fmt(table.T, tail)

  grab = pl.kernel(
      functools.partial(_emb_body, S, TB),
      out_type=jax.ShapeDtypeStruct((S, 8, T // TW, 8, TW), jnp.float32),
      mesh=mesh,
      scratch_types=[
          pltpu.VMEM((S, TB * TW), jnp.int32),
          pltpu.VMEM((NBUF, TW, TW), jnp.float32),
          pltpu.VMEM((NBUF, EMB_DIM, YS), jnp.float32),
          pltpu.SemaphoreType.DMA,
          pltpu.SemaphoreType.DMA,
          pltpu.SemaphoreType.DMA,
          pltpu.SemaphoreType.DMA,
      ],
      compiler_params=pltpu.CompilerParams(
          use_tc_tiling_on_sc=False, needs_layout_passes=False),
  )
  q = grab(idx_t, lin)
  return q.transpose(2, 4, 0, 1, 3).reshape(T, S, EMB_DIM)


# speedup vs baseline: 4.2664x; 1.2980x over previous
import functools

import jax
import jax.numpy as jnp
from jax import lax
from jax.experimental import pallas as pl
from jax.experimental.pallas import tpu as pltpu
from jax.experimental.pallas import tpu_sc as plsc

VOCAB = 1000000
EMB_DIM = 64

NC = 2
NS = 16
NW = NC * NS

TW = 128
NBUF = 2
YS = TW + 5

NCOL = (VOCAB + TW - 1) // TW
FULL_COL = VOCAB // TW


def _fmt_body(tbl_hbm, tail_hbm, lin_hbm, x_v, y_v, g0, g1, w0, w1):
  gsem = (g0, g1)
  wsem = (w0, w1)
  wid = lax.axis_index("s") * NC + lax.axis_index("c")
  n_units = (NCOL + NW - 1) // NW

  def fire_in(u, b):
    c = u * NW + wid

    @pl.when(c < FULL_COL)
    def _():
      pltpu.async_copy(tbl_hbm.at[:, pl.ds(c * TW, TW)], x_v.at[b], gsem[b])

    @pl.when(c == FULL_COL)
    def _():
      pltpu.async_copy(tail_hbm, x_v.at[b], gsem[b])

  def wait_in(u, b):
    c = u * NW + wid

    @pl.when(c < FULL_COL)
    def _():
      pltpu.make_async_copy(
          tbl_hbm.at[:, pl.ds(0, TW)], x_v.at[b], gsem[b]).wait()

    @pl.when(c == FULL_COL)
    def _():
      pltpu.make_async_copy(tail_hbm, x_v.at[b], gsem[b]).wait()

  iota16 = lax.iota(jnp.int32, 16)
  rots = [(iota16 + k) & 15 for k in range(16)]

  def transpose(b):
    x = x_v.at[b]
    y = y_v.at[b]

    @plsc.parallel_loop(0, (EMB_DIM // 16) * (TW // 16), unroll=2)
    def blk_body(i):
      vb16 = (i & 7) * 16
      eb16 = (i >> 3) * 16
      ev = iota16 + eb16
      for k in range(16):
        vv = rots[k] + vb16
        cv = ((vv & 1) << 6) + ev
        plsc.store_scatter(
            y, [vv >> 1, cv], plsc.load_gather(x, [ev, vv]))

  def fire_out(u, b):
    c = u * NW + wid

    @pl.when(c < FULL_COL)
    def _():
      pltpu.async_copy(
          y_v.at[b], lin_hbm.at[pl.ds(c * (TW // 2), TW // 2)], wsem[b])

    @pl.when(c == FULL_COL)
    def _():
      pltpu.async_copy(
          y_v.at[b, pl.ds(0, TW // 4)],
          lin_hbm.at[pl.ds(FULL_COL * (TW // 2), TW // 4)], wsem[b])

  def wait_out(u, b):
    c = u * NW + wid

    @pl.when(c < FULL_COL)
    def _():
      pltpu.make_async_copy(
          y_v.at[b], lin_hbm.at[pl.ds(0, TW // 2)], wsem[b]).wait()

    @pl.when(c == FULL_COL)
    def _():
      pltpu.make_async_copy(
          y_v.at[b, pl.ds(0, TW // 4)],
          lin_hbm.at[pl.ds(0, TW // 4)], wsem[b]).wait()

  fire_in(0, 0)

  def pair(g, _):
    for b in range(NBUF):
      u = g * NBUF + b

      @pl.when((u + 1) * NW + wid < NCOL)
      def _():
        fire_in(u + 1, 1 - b)

      @pl.when(u * NW + wid < NCOL)
      def _():
        wait_in(u, b)

        @pl.when(u >= NBUF)
        def _():
          wait_out(u - NBUF, b)

        transpose(b)
        fire_out(u, b)

    return 0

  n_outer = (n_units + NBUF - 1) // NBUF
  lax.fori_loop(0, n_outer, pair, 0)
  lv = (NCOL - 1 - wid) // NW
  for k in range(NBUF):
    u = lv - k
    for bb in range(NBUF):
      @pl.when(lax.rem(u, NBUF) == bb)
      def _():
        wait_out(u, bb)


def _emb_body(S, TB, idx_hbm, table_hbm, out_hbm, idx_v, rows_v, tiles_v,
              g0, g1, w0, w1):
  gsem = (g0, g1)
  wsem = (w0, w1)
  wid = lax.axis_index("s") * NC + lax.axis_index("c")
  t0 = wid * (TB * TW)
  n_units = S * TB

  pltpu.sync_copy(idx_hbm.at[:, pl.ds(t0, TB * TW)], idx_v)

  def fire_gather(u, b):
    s = u // TB
    q = u % TB
    pltpu.async_copy(
        table_hbm.at[idx_v.at[s, pl.ds(q * TW, TW)]],
        rows_v.at[b],
        gsem[b],
    )

  def wait_gather(b):
    pltpu.make_async_copy(
        table_hbm.at[pl.ds(0, TW)], rows_v.at[b], gsem[b]).wait()

  def fire_write(u, b):
    s = u // TB
    t_hi = wid * TB + u % TB
    for ehi in range(8):
      pltpu.async_copy(
          tiles_v.at[b, pl.ds(ehi * 8, 8), pl.ds(0, TW)],
          out_hbm.at[s, ehi, t_hi], wsem[b])

  def wait_write(b):
    for ehi in range(8):
      pltpu.make_async_copy(
          tiles_v.at[b, pl.ds(ehi * 8, 8), pl.ds(0, TW)],
          out_hbm.at[0, ehi, 0], wsem[b]).wait()

  iota = lax.iota(jnp.int32, 16)
  evs = [iota + eb * 16 for eb in range(EMB_DIM // 16)]

  def transpose(b):
    rows = rows_v.at[b]
    tiles = tiles_v.at[b]

    @plsc.parallel_loop(0, TW, unroll=4)
    def t_body(t):
      cv = lax.broadcast(t, (16,))
      for eb in range(EMB_DIM // 16):
        plsc.store_scatter(
            tiles, [evs[eb], cv], rows[t, pl.ds(eb * 16, 16)])

  fire_gather(0, 0)

  def pair(g, _):
    for b in range(NBUF):
      u = g * NBUF + b

      @pl.when(u + 1 < n_units)
      def _():
        fire_gather(u + 1, 1 - b)

      wait_gather(b)

      @pl.when(u >= NBUF)
      def _():
        wait_write(b)

      transpose(b)
      fire_write(u, b)

    return 0

  lax.fori_loop(0, n_units // NBUF, pair, 0)
  for b in range(NBUF):
    wait_write(b)


def kernel(indices, table):
  T, S = indices.shape
  assert T % (NW * TW) == 0 and EMB_DIM == 64
  TB = T // (NW * TW)
  idx_t = indices.T.astype(jnp.int32)

  mesh = plsc.VectorSubcoreMesh(core_axis_name="c", subcore_axis_name="s")

  fmt = pl.kernel(
      _fmt_body,
      out_type=jax.ShapeDtypeStruct((VOCAB // 2, TW), jnp.float32),
      mesh=mesh,
      scratch_types=[
          pltpu.VMEM((NBUF, EMB_DIM, TW), jnp.float32),
          pltpu.VMEM((NBUF, TW // 2, TW), jnp.float32),
          pltpu.SemaphoreType.DMA,
          pltpu.SemaphoreType.DMA,
          pltpu.SemaphoreType.DMA,
          pltpu.SemaphoreType.DMA,
      ],
      compiler_params=pltpu.CompilerParams(
          use_tc_tiling_on_sc=True, needs_layout_passes=False),
  )
  tail = jnp.pad(table[FULL_COL * TW:].T, ((0, 0), (0, TW - (VOCAB - FULL_COL * TW))))
  lin = fmt(table.T, tail)

  grab = pl.kernel(
      functools.partial(_emb_body, S, TB),
      out_type=jax.ShapeDtypeStruct((S, 8, T // TW, 8, TW), jnp.float32),
      mesh=mesh,
      scratch_types=[
          pltpu.VMEM((S, TB * TW), jnp.int32),
          pltpu.VMEM((NBUF, TW, EMB_DIM), jnp.float32),
          pltpu.VMEM((NBUF, EMB_DIM, YS), jnp.float32),
          pltpu.SemaphoreType.DMA,
          pltpu.SemaphoreType.DMA,
          pltpu.SemaphoreType.DMA,
          pltpu.SemaphoreType.DMA,
      ],
      compiler_params=pltpu.CompilerParams(
          use_tc_tiling_on_sc=False, needs_layout_passes=False),
  )
  q = grab(idx_t, lin.reshape(VOCAB, EMB_DIM))
  return q.transpose(2, 4, 0, 1, 3).reshape(T, S, EMB_DIM)
